# baseline (speedup 1.0000x reference)
import numpy as np
import jax
import jax.numpy as jnp
from jax import lax
from jax.experimental import pallas as pl

_NUM_TIME = 8
_NUM_LABEL = 4
_SPLIT = 6
_D = 32
_NSEG = _NUM_TIME * _NUM_LABEL

_RA = 5000
_RC = 5000


def _seg_body(x_ref, seg_ref, o_ref):
    i = pl.program_id(0)
    xx = x_ref[...]
    sg = seg_ref[...]
    r = xx.shape[0]
    x_aug = jnp.concatenate([xx, jnp.ones((r, 8), jnp.float32)], axis=1)
    ohs = (sg == lax.broadcasted_iota(jnp.int32, (r, _NSEG), 1)).astype(jnp.float32)
    d = lax.dot_general(ohs, x_aug, (((0,), (0,)), ((), ())),
                        preferred_element_type=jnp.float32)

    @pl.when(i == 0)
    def _():
        o_ref[...] = d

    @pl.when(i > 0)
    def _():
        o_ref[...] += d


def _apply_body(x_ref, seg_ref, rs_ref, w2_ref, o_ref):
    xx = x_ref[...]
    sg = seg_ref[...]
    r = xx.shape[0]
    xb = xx.astype(jnp.bfloat16)
    lab = sg % _NUM_LABEL
    tt = sg // _NUM_LABEL
    labb = lab.astype(jnp.bfloat16)
    ttb = tt.astype(jnp.bfloat16)

    p4 = (lax.broadcasted_iota(jnp.int32, (r, _NUM_LABEL * _D), 1)
          // _D).astype(jnp.bfloat16)
    mask4 = labb == p4
    xt4 = jnp.concatenate([xb] * _NUM_LABEL, axis=1)
    u4 = jnp.where(mask4, xt4, jnp.bfloat16(0))
    v = lax.dot_general(u4, rs_ref[...], (((1,), (0,)), ((), ())),
                        preferred_element_type=jnp.float32
                        ).astype(jnp.bfloat16)

    p6 = (lax.broadcasted_iota(jnp.int32, (r, _SPLIT * _D), 1)
          // _D).astype(jnp.bfloat16)
    vz = jnp.where(ttb == p6, v, jnp.bfloat16(0))
    w = vz[:, :_D]
    for tslice in range(1, _SPLIT):
        w = w + vz[:, tslice * _D:(tslice + 1) * _D]

    u2 = jnp.where(mask4, jnp.concatenate([w] * _NUM_LABEL, axis=1),
                   jnp.bfloat16(0))
    ohs = (sg == lax.broadcasted_iota(jnp.int32, (r, _NSEG), 1)
           ).astype(jnp.bfloat16)
    z160 = jnp.concatenate([u2, ohs], axis=1)
    y = lax.dot_general(z160, w2_ref[...], (((1,), (0,)), ((), ())),
                        preferred_element_type=jnp.float32)
    o_ref[...] = jnp.where(tt < _SPLIT, y, xx)


def _transform_bank(prev_x, labels, P):
    covs = []
    for y in range(_NUM_LABEL):
        mask = (labels == y).astype(prev_x.dtype)
        n = mask.sum()
        mean = (prev_x * mask[:, None]).sum(0) / n
        xc = (prev_x - mean[None, :]) * mask[:, None]
        covs.append(xc.T @ xc / (n - 1.0))
    prev_cov = jnp.stack(covs)

    t = np.arange(_NUM_TIME)
    cond = np.abs(t[None, :] - t[:, None]) > np.minimum(_NUM_TIME - 1 - t, t)[:, None]
    f2 = jnp.asarray(np.where(cond, 2.0, 1.0), dtype=jnp.float32)
    f4 = jnp.asarray(np.where(cond, 4.0, 1.0), dtype=jnp.float32)
    denom = jnp.einsum('atbs,ts->at', P, f2)
    denom2 = denom * denom
    temp = jnp.einsum('atbs,ts->atb', P, f4) / denom2[:, :, None]
    current_cov = jnp.einsum('atb,bij->atij', temp, prev_cov)

    lall, qall = jnp.linalg.eigh(current_cov)
    l_max = lall[:, _NUM_TIME - 1]
    q_max = qall[:, _NUM_TIME - 1]
    left = q_max * jnp.sqrt(l_max)[:, None, :]
    right = qall * (1.0 / jnp.sqrt(lall))[:, :, None, :]
    a = jnp.einsum('yik,ytjk->ytij', left, right)
    return left, right, a


def kernel(prev_x, x, labels, times, P):
    n = x.shape[0]
    seg = (times.astype(jnp.int32) * _NUM_LABEL
           + labels.astype(jnp.int32)).reshape(n, 1)

    left, right, a = _transform_bank(prev_x, labels, P)

    o2 = pl.pallas_call(
        _seg_body,
        grid=(n // _RA,),
        in_specs=[
            pl.BlockSpec((_RA, _D), lambda i: (i, 0)),
            pl.BlockSpec((_RA, 1), lambda i: (i, 0)),
        ],
        out_specs=pl.BlockSpec((_NSEG, 40), lambda i: (0, 0)),
        out_shape=jax.ShapeDtypeStruct((_NSEG, 40), jnp.float32),
    )(x, seg)

    mu = o2[:, :_D] / jnp.maximum(o2[:, _D], 1.0)[:, None]
    a_seg = jnp.transpose(a, (1, 0, 2, 3)).reshape(_NSEG, _D, _D)
    b_seg = mu - jnp.einsum('sij,sj->si', a_seg, mu)
    rs = jnp.transpose(right[:, :_SPLIT], (0, 2, 1, 3)).reshape(
        _NUM_LABEL * _D, _SPLIT * _D).astype(jnp.bfloat16)
    ls = jnp.transpose(left, (0, 2, 1)).reshape(_NUM_LABEL * _D, _D)
    w2 = jnp.concatenate([ls, b_seg], axis=0).astype(jnp.bfloat16)

    out = pl.pallas_call(
        _apply_body,
        grid=(n // _RC,),
        in_specs=[
            pl.BlockSpec((_RC, _D), lambda i: (i, 0)),
            pl.BlockSpec((_RC, 1), lambda i: (i, 0)),
            pl.BlockSpec((_NUM_LABEL * _D, _SPLIT * _D), lambda i: (0, 0)),
            pl.BlockSpec((_NUM_LABEL * _D + _NSEG, _D), lambda i: (0, 0)),
        ],
        out_specs=pl.BlockSpec((_RC, _D), lambda i: (i, 0)),
        out_shape=jax.ShapeDtypeStruct((n, _D), jnp.float32),
    )(x, seg, rs, w2)
    return out


# speedup vs baseline: 1.0025x; 1.0025x over previous
import numpy as np
import jax
import jax.numpy as jnp
from jax import lax
from jax.experimental import pallas as pl

_NUM_TIME = 8
_NUM_LABEL = 4
_SPLIT = 6
_D = 32
_NSEG = _NUM_TIME * _NUM_LABEL

_RA = 5000
_RC = 5000


def _seg_body(x_ref, seg_ref, o_ref):
    i = pl.program_id(0)
    xx = x_ref[...]
    sg = seg_ref[...]
    r = xx.shape[0]
    x_aug = jnp.concatenate([xx, jnp.ones((r, 8), jnp.float32)], axis=1)
    ohs = (sg == lax.broadcasted_iota(jnp.int32, (r, _NSEG), 1)).astype(jnp.float32)
    d = lax.dot_general(ohs, x_aug, (((0,), (0,)), ((), ())),
                        preferred_element_type=jnp.float32)

    @pl.when(i == 0)
    def _():
        o_ref[...] = d

    @pl.when(i > 0)
    def _():
        o_ref[...] += d


def _apply_body(x_ref, seg_ref, w_ref, o_ref):
    xx = x_ref[...]
    sg = seg_ref[...]
    r = xx.shape[0]
    xb = xx.astype(jnp.bfloat16)
    lab = sg % _NUM_LABEL
    tt = sg // _NUM_LABEL
    labb = lab.astype(jnp.bfloat16)
    sgb = sg.astype(jnp.bfloat16)

    p4 = (lax.broadcasted_iota(jnp.int32, (r, _NUM_LABEL * _D), 1)
          // _D).astype(jnp.bfloat16)
    x128 = jnp.concatenate([xb] * _NUM_LABEL, axis=1)
    u4 = jnp.where(labb == p4, x128, jnp.bfloat16(0))
    blocks = []
    for t in range(_SPLIT):
        ft = (tt == t).astype(jnp.bfloat16)
        blocks.append(u4 * ft)
    p128 = lax.broadcasted_iota(jnp.int32, (r, 128), 1).astype(jnp.bfloat16)
    ohs128 = (sgb == p128).astype(jnp.bfloat16)
    z = jnp.concatenate(blocks + [ohs128], axis=1)
    y = lax.dot_general(z, w_ref[...], (((1,), (0,)), ((), ())),
                        preferred_element_type=jnp.float32)
    o_ref[...] = jnp.where(tt < _SPLIT, y, xx)


def _transform_bank(prev_x, labels, P):
    covs = []
    for y in range(_NUM_LABEL):
        mask = (labels == y).astype(prev_x.dtype)
        n = mask.sum()
        mean = (prev_x * mask[:, None]).sum(0) / n
        xc = (prev_x - mean[None, :]) * mask[:, None]
        covs.append(xc.T @ xc / (n - 1.0))
    prev_cov = jnp.stack(covs)

    t = np.arange(_NUM_TIME)
    cond = np.abs(t[None, :] - t[:, None]) > np.minimum(_NUM_TIME - 1 - t, t)[:, None]
    f2 = jnp.asarray(np.where(cond, 2.0, 1.0), dtype=jnp.float32)
    f4 = jnp.asarray(np.where(cond, 4.0, 1.0), dtype=jnp.float32)
    denom = jnp.einsum('atbs,ts->at', P, f2)
    denom2 = denom * denom
    temp = jnp.einsum('atbs,ts->atb', P, f4) / denom2[:, :, None]
    current_cov = jnp.einsum('atb,bij->atij', temp, prev_cov)

    lall, qall = jnp.linalg.eigh(current_cov)
    l_max = lall[:, _NUM_TIME - 1]
    q_max = qall[:, _NUM_TIME - 1]
    left = q_max * jnp.sqrt(l_max)[:, None, :]
    right = qall * (1.0 / jnp.sqrt(lall))[:, :, None, :]
    a = jnp.einsum('yik,ytjk->ytij', left, right)
    return left, right, a


def kernel(prev_x, x, labels, times, P):
    n = x.shape[0]
    seg = (times.astype(jnp.int32) * _NUM_LABEL
           + labels.astype(jnp.int32)).reshape(n, 1)

    left, right, a = _transform_bank(prev_x, labels, P)

    o2 = pl.pallas_call(
        _seg_body,
        grid=(n // _RA,),
        in_specs=[
            pl.BlockSpec((_RA, _D), lambda i: (i, 0)),
            pl.BlockSpec((_RA, 1), lambda i: (i, 0)),
        ],
        out_specs=pl.BlockSpec((_NSEG, 40), lambda i: (0, 0)),
        out_shape=jax.ShapeDtypeStruct((_NSEG, 40), jnp.float32),
    )(x, seg)

    mu = o2[:, :_D] / jnp.maximum(o2[:, _D], 1.0)[:, None]
    a_seg = jnp.transpose(a, (1, 0, 2, 3)).reshape(_NSEG, _D, _D)
    b_seg = mu - jnp.einsum('sij,sj->si', a_seg, mu)
    w1 = jnp.transpose(a[:, :_SPLIT], (1, 0, 3, 2)).reshape(
        _SPLIT * _NUM_LABEL * _D, _D)
    b_pad = jnp.concatenate(
        [b_seg, jnp.zeros((128 - _NSEG, _D), jnp.float32)], axis=0)
    w_bank = jnp.concatenate([w1, b_pad], axis=0).astype(jnp.bfloat16)

    kw = _SPLIT * _NUM_LABEL * _D + 128
    out = pl.pallas_call(
        _apply_body,
        grid=(n // _RC,),
        in_specs=[
            pl.BlockSpec((_RC, _D), lambda i: (i, 0)),
            pl.BlockSpec((_RC, 1), lambda i: (i, 0)),
            pl.BlockSpec((kw, _D), lambda i: (0, 0)),
        ],
        out_specs=pl.BlockSpec((_RC, _D), lambda i: (i, 0)),
        out_shape=jax.ShapeDtypeStruct((n, _D), jnp.float32),
    )(x, seg, w_bank)
    return out


# speedup vs baseline: 1.0415x; 1.0388x over previous
import numpy as np
import jax
import jax.numpy as jnp
from jax import lax
from jax.experimental import pallas as pl

_NUM_TIME = 8
_NUM_LABEL = 4
_SPLIT = 6
_D = 32
_NSEG = _NUM_TIME * _NUM_LABEL

_RA = 5000
_RC = 5000


def _seg_body(x_ref, seg_ref, o_ref):
    i = pl.program_id(0)
    xx = x_ref[...]
    sg = seg_ref[...]
    r = xx.shape[0]
    x_aug = jnp.concatenate([xx, jnp.ones((r, 8), jnp.float32)], axis=1)
    ohs = (sg == lax.broadcasted_iota(jnp.int32, (r, _NSEG), 1)).astype(jnp.float32)
    d = lax.dot_general(ohs, x_aug, (((0,), (0,)), ((), ())),
                        preferred_element_type=jnp.float32)

    @pl.when(i == 0)
    def _():
        o_ref[...] = d

    @pl.when(i > 0)
    def _():
        o_ref[...] += d


def _apply_body(x_ref, seg_ref, w_ref, o_ref):
    xx = x_ref[...]
    sg = seg_ref[...]
    r = xx.shape[0]
    xb = xx.astype(jnp.bfloat16)
    tt = sg // _NUM_LABEL
    sgb = sg.astype(jnp.bfloat16)

    x128 = jnp.concatenate([xb] * _NUM_LABEL, axis=1)
    xt768 = jnp.concatenate([x128] * _SPLIT, axis=1)
    patt = (lax.broadcasted_iota(jnp.int32, (r, _SPLIT * _NUM_LABEL * _D), 1)
            // _D).astype(jnp.bfloat16)
    z768 = jnp.where(sgb == patt, xt768, jnp.bfloat16(0))
    p128 = lax.broadcasted_iota(jnp.int32, (r, 128), 1).astype(jnp.bfloat16)
    ohs128 = (sgb == p128).astype(jnp.bfloat16)
    z = jnp.concatenate([z768, ohs128], axis=1)
    y = lax.dot_general(z, w_ref[...], (((1,), (0,)), ((), ())),
                        preferred_element_type=jnp.float32)
    o_ref[...] = jnp.where(tt < _SPLIT, y, xx)


def _transform_bank(prev_x, labels, P):
    covs = []
    for y in range(_NUM_LABEL):
        mask = (labels == y).astype(prev_x.dtype)
        n = mask.sum()
        mean = (prev_x * mask[:, None]).sum(0) / n
        xc = (prev_x - mean[None, :]) * mask[:, None]
        covs.append(xc.T @ xc / (n - 1.0))
    prev_cov = jnp.stack(covs)

    t = np.arange(_NUM_TIME)
    cond = np.abs(t[None, :] - t[:, None]) > np.minimum(_NUM_TIME - 1 - t, t)[:, None]
    f2 = jnp.asarray(np.where(cond, 2.0, 1.0), dtype=jnp.float32)
    f4 = jnp.asarray(np.where(cond, 4.0, 1.0), dtype=jnp.float32)
    denom = jnp.einsum('atbs,ts->at', P, f2)
    denom2 = denom * denom
    temp = jnp.einsum('atbs,ts->atb', P, f4) / denom2[:, :, None]
    current_cov = jnp.einsum('atb,bij->atij', temp, prev_cov)

    lall, qall = jnp.linalg.eigh(current_cov)
    l_max = lall[:, _NUM_TIME - 1]
    q_max = qall[:, _NUM_TIME - 1]
    left = q_max * jnp.sqrt(l_max)[:, None, :]
    right = qall * (1.0 / jnp.sqrt(lall))[:, :, None, :]
    a = jnp.einsum('yik,ytjk->ytij', left, right)
    return left, right, a


def kernel(prev_x, x, labels, times, P):
    n = x.shape[0]
    seg = (times.astype(jnp.int32) * _NUM_LABEL
           + labels.astype(jnp.int32)).reshape(n, 1)

    left, right, a = _transform_bank(prev_x, labels, P)

    o2 = pl.pallas_call(
        _seg_body,
        grid=(n // _RA,),
        in_specs=[
            pl.BlockSpec((_RA, _D), lambda i: (i, 0)),
            pl.BlockSpec((_RA, 1), lambda i: (i, 0)),
        ],
        out_specs=pl.BlockSpec((_NSEG, 40), lambda i: (0, 0)),
        out_shape=jax.ShapeDtypeStruct((_NSEG, 40), jnp.float32),
    )(x, seg)

    mu = o2[:, :_D] / jnp.maximum(o2[:, _D], 1.0)[:, None]
    a_seg = jnp.transpose(a, (1, 0, 2, 3)).reshape(_NSEG, _D, _D)
    b_seg = mu - jnp.einsum('sij,sj->si', a_seg, mu)
    w1 = jnp.transpose(a[:, :_SPLIT], (1, 0, 3, 2)).reshape(
        _SPLIT * _NUM_LABEL * _D, _D)
    b_pad = jnp.concatenate(
        [b_seg, jnp.zeros((128 - _NSEG, _D), jnp.float32)], axis=0)
    w_bank = jnp.concatenate([w1, b_pad], axis=0).astype(jnp.bfloat16)

    kw = _SPLIT * _NUM_LABEL * _D + 128
    out = pl.pallas_call(
        _apply_body,
        grid=(n // _RC,),
        in_specs=[
            pl.BlockSpec((_RC, _D), lambda i: (i, 0)),
            pl.BlockSpec((_RC, 1), lambda i: (i, 0)),
            pl.BlockSpec((kw, _D), lambda i: (0, 0)),
        ],
        out_specs=pl.BlockSpec((_RC, _D), lambda i: (i, 0)),
        out_shape=jax.ShapeDtypeStruct((n, _D), jnp.float32),
    )(x, seg, w_bank)
    return out


# speedup vs baseline: 1.0590x; 1.0168x over previous
import functools

import numpy as np
import jax
import jax.numpy as jnp
from jax import lax
from jax.experimental import pallas as pl
from jax.experimental.pallas import tpu as pltpu, tpu_sc as plsc

_NUM_TIME = 8
_NUM_LABEL = 4
_SPLIT = 6
_D = 32
_NSEG = _NUM_TIME * _NUM_LABEL

_RA = 5000
_RC = 5000


_SC_NW = 25
_SC_ROWS = 4000
_SC_CH = 400
_SC_SW = 48
_SC_ACC = 33 * _SC_SW


def _sc_segsum(x, segf):
    mesh = plsc.VectorSubcoreMesh(core_axis_name="c", subcore_axis_name="s")

    @functools.partial(
        pl.kernel, mesh=mesh,
        out_type=jax.ShapeDtypeStruct((_SC_NW, _SC_ACC), jnp.float32),
        scratch_types=[
            pltpu.VMEM((_SC_CH, _D), jnp.float32),
            pltpu.VMEM((_SC_CH,), jnp.int32),
            pltpu.VMEM((_SC_ACC,), jnp.float32),
        ],
    )
    def k(x_hbm, seg_hbm, out_hbm, xv, segv, acc):
        wid = lax.axis_index("s") * 2 + lax.axis_index("c")

        @pl.when(wid < _SC_NW)
        def _():
            for z in range(_SC_ACC // 16):
                acc[pl.ds(z * 16, 16)] = jnp.zeros((16,), jnp.float32)
            ones16 = jnp.ones((16,), jnp.float32)

            def chunk_body(c, carry):
                base = wid * _SC_ROWS + c * _SC_CH
                pltpu.sync_copy(x_hbm.at[pl.ds(base, _SC_CH)], xv)
                pltpu.sync_copy(seg_hbm.at[pl.ds(base, _SC_CH)], segv)

                def row_body(g, carry2):
                    sv = segv[pl.ds(g * 16, 16)]
                    for u in range(16):
                        row = g * 16 + u
                        sb = sv[u] * _SC_SW
                        a0 = acc[pl.ds(sb, 16)]
                        acc[pl.ds(sb, 16)] = a0 + xv[row, pl.ds(0, 16)]
                        a1 = acc[pl.ds(sb + 16, 16)]
                        acc[pl.ds(sb + 16, 16)] = a1 + xv[row, pl.ds(16, 16)]
                        a2 = acc[pl.ds(sb + 32, 16)]
                        acc[pl.ds(sb + 32, 16)] = a2 + ones16
                    return carry2

                lax.fori_loop(0, _SC_CH // 16, row_body, 0)
                return carry

            lax.fori_loop(0, _SC_ROWS // _SC_CH, chunk_body, 0)
            pltpu.sync_copy(acc, out_hbm.at[wid])

    return k(x, segf)


def _apply_body(x_ref, seg_ref, w_ref, o_ref):
    xx = x_ref[...]
    sg = seg_ref[...]
    r = xx.shape[0]
    xb = xx.astype(jnp.bfloat16)
    tt = sg // _NUM_LABEL
    sgb = sg.astype(jnp.bfloat16)

    x128 = jnp.concatenate([xb] * _NUM_LABEL, axis=1)
    xt768 = jnp.concatenate([x128] * _SPLIT, axis=1)
    patt = (lax.broadcasted_iota(jnp.int32, (r, _SPLIT * _NUM_LABEL * _D), 1)
            // _D).astype(jnp.bfloat16)
    z768 = jnp.where(sgb == patt, xt768, jnp.bfloat16(0))
    p128 = lax.broadcasted_iota(jnp.int32, (r, 128), 1).astype(jnp.bfloat16)
    ohs128 = (sgb == p128).astype(jnp.bfloat16)
    z = jnp.concatenate([z768, ohs128], axis=1)
    y = lax.dot_general(z, w_ref[...], (((1,), (0,)), ((), ())),
                        preferred_element_type=jnp.float32)
    o_ref[...] = jnp.where(tt < _SPLIT, y, xx)


def _transform_bank(prev_x, labels, P):
    covs = []
    for y in range(_NUM_LABEL):
        mask = (labels == y).astype(prev_x.dtype)
        n = mask.sum()
        mean = (prev_x * mask[:, None]).sum(0) / n
        xc = (prev_x - mean[None, :]) * mask[:, None]
        covs.append(xc.T @ xc / (n - 1.0))
    prev_cov = jnp.stack(covs)

    t = np.arange(_NUM_TIME)
    cond = np.abs(t[None, :] - t[:, None]) > np.minimum(_NUM_TIME - 1 - t, t)[:, None]
    f2 = jnp.asarray(np.where(cond, 2.0, 1.0), dtype=jnp.float32)
    f4 = jnp.asarray(np.where(cond, 4.0, 1.0), dtype=jnp.float32)
    denom = jnp.einsum('atbs,ts->at', P, f2)
    denom2 = denom * denom
    temp = jnp.einsum('atbs,ts->atb', P, f4) / denom2[:, :, None]
    current_cov = jnp.einsum('atb,bij->atij', temp, prev_cov)

    lall, qall = jnp.linalg.eigh(current_cov)
    l_max = lall[:, _NUM_TIME - 1]
    q_max = qall[:, _NUM_TIME - 1]
    left = q_max * jnp.sqrt(l_max)[:, None, :]
    right = qall * (1.0 / jnp.sqrt(lall))[:, :, None, :]
    a = jnp.einsum('yik,ytjk->ytij', left, right)
    return left, right, a


def kernel(prev_x, x, labels, times, P):
    n = x.shape[0]
    segf = times.astype(jnp.int32) * _NUM_LABEL + labels.astype(jnp.int32)
    seg = segf.reshape(n, 1)

    left, right, a = _transform_bank(prev_x, labels, P)

    o2p = _sc_segsum(x, segf)
    o2 = o2p.sum(0)[:_NSEG * _SC_SW].reshape(_NSEG, _SC_SW)
    mu = o2[:, :_D] / jnp.maximum(o2[:, _D], 1.0)[:, None]
    a_seg = jnp.transpose(a, (1, 0, 2, 3)).reshape(_NSEG, _D, _D)
    b_seg = mu - jnp.einsum('sij,sj->si', a_seg, mu)
    w1 = jnp.transpose(a[:, :_SPLIT], (1, 0, 3, 2)).reshape(
        _SPLIT * _NUM_LABEL * _D, _D)
    b_pad = jnp.concatenate(
        [b_seg, jnp.zeros((128 - _NSEG, _D), jnp.float32)], axis=0)
    w_bank = jnp.concatenate([w1, b_pad], axis=0).astype(jnp.bfloat16)

    kw = _SPLIT * _NUM_LABEL * _D + 128
    out = pl.pallas_call(
        _apply_body,
        grid=(n // _RC,),
        in_specs=[
            pl.BlockSpec((_RC, _D), lambda i: (i, 0)),
            pl.BlockSpec((_RC, 1), lambda i: (i, 0)),
            pl.BlockSpec((kw, _D), lambda i: (0, 0)),
        ],
        out_specs=pl.BlockSpec((_RC, _D), lambda i: (i, 0)),
        out_shape=jax.ShapeDtypeStruct((n, _D), jnp.float32),
    )(x, seg, w_bank)
    return out


# speedup vs baseline: 1.0591x; 1.0002x over previous
import functools

import numpy as np
import jax
import jax.numpy as jnp
from jax import lax
from jax.experimental import pallas as pl
from jax.experimental.pallas import tpu as pltpu, tpu_sc as plsc

_NUM_TIME = 8
_NUM_LABEL = 4
_SPLIT = 6
_D = 32
_NSEG = _NUM_TIME * _NUM_LABEL

_RC = 5000


_SC_NW = 25
_SC_ROWS = 4000
_SC_CH = 400
_SC_SW = 48
_SC_ACC = 33 * _SC_SW


def _sc_segsum(x, segf):
    mesh = plsc.VectorSubcoreMesh(core_axis_name="c", subcore_axis_name="s")

    @functools.partial(
        pl.kernel, mesh=mesh,
        out_type=jax.ShapeDtypeStruct((_SC_NW, _SC_ACC), jnp.float32),
        scratch_types=[
            pltpu.VMEM((_SC_CH, _D), jnp.float32),
            pltpu.VMEM((_SC_CH,), jnp.int32),
            pltpu.VMEM((_SC_ACC,), jnp.float32),
        ],
    )
    def k(x_hbm, seg_hbm, out_hbm, xv, segv, acc):
        wid = lax.axis_index("s") * 2 + lax.axis_index("c")

        @pl.when(wid < _SC_NW)
        def _():
            for z in range(_SC_ACC // 16):
                acc[pl.ds(z * 16, 16)] = jnp.zeros((16,), jnp.float32)
            ones16 = jnp.ones((16,), jnp.float32)

            def chunk_body(c, carry):
                base = wid * _SC_ROWS + c * _SC_CH
                pltpu.sync_copy(x_hbm.at[pl.ds(base, _SC_CH)], xv)
                pltpu.sync_copy(seg_hbm.at[pl.ds(base, _SC_CH)], segv)

                def row_body(g, carry2):
                    sv = segv[pl.ds(g * 16, 16)]
                    for u in range(16):
                        row = g * 16 + u
                        sb = sv[u] * _SC_SW
                        a0 = acc[pl.ds(sb, 16)]
                        acc[pl.ds(sb, 16)] = a0 + xv[row, pl.ds(0, 16)]
                        a1 = acc[pl.ds(sb + 16, 16)]
                        acc[pl.ds(sb + 16, 16)] = a1 + xv[row, pl.ds(16, 16)]
                        a2 = acc[pl.ds(sb + 32, 16)]
                        acc[pl.ds(sb + 32, 16)] = a2 + ones16
                    return carry2

                lax.fori_loop(0, _SC_CH // 16, row_body, 0)
                return carry

            lax.fori_loop(0, _SC_ROWS // _SC_CH, chunk_body, 0)
            pltpu.sync_copy(acc, out_hbm.at[wid])

    return k(x, segf)


def _apply_body(x_ref, seg_ref, w_ref, o_ref):
    xx = x_ref[...]
    sg = seg_ref[...]
    r = xx.shape[0]
    xb = xx.astype(jnp.bfloat16)
    tt = sg // _NUM_LABEL
    sgb = sg.astype(jnp.bfloat16)

    x128 = jnp.concatenate([xb] * _NUM_LABEL, axis=1)
    xt768 = jnp.concatenate([x128] * _SPLIT, axis=1)
    patt = (lax.broadcasted_iota(jnp.int32, (r, _SPLIT * _NUM_LABEL * _D), 1)
            // _D).astype(jnp.bfloat16)
    z768 = jnp.where(sgb == patt, xt768, jnp.bfloat16(0))
    p128 = lax.broadcasted_iota(jnp.int32, (r, 128), 1).astype(jnp.bfloat16)
    ohs128 = (sgb == p128).astype(jnp.bfloat16)
    z = jnp.concatenate([z768, ohs128], axis=1)
    y = lax.dot_general(z, w_ref[...], (((1,), (0,)), ((), ())),
                        preferred_element_type=jnp.float32)
    o_ref[...] = jnp.where(tt < _SPLIT, y, xx)


def _transform_bank(prev_x, labels, P):
    covs = []
    for y in range(_NUM_LABEL):
        mask = (labels == y).astype(prev_x.dtype)
        n = mask.sum()
        mean = (prev_x * mask[:, None]).sum(0) / n
        xc = (prev_x - mean[None, :]) * mask[:, None]
        covs.append(xc.T @ xc / (n - 1.0))
    prev_cov = jnp.stack(covs)

    t = np.arange(_NUM_TIME)
    cond = np.abs(t[None, :] - t[:, None]) > np.minimum(_NUM_TIME - 1 - t, t)[:, None]
    f2 = jnp.asarray(np.where(cond, 2.0, 1.0), dtype=jnp.float32)
    f4 = jnp.asarray(np.where(cond, 4.0, 1.0), dtype=jnp.float32)
    denom = jnp.einsum('atbs,ts->at', P, f2)
    denom2 = denom * denom
    temp = jnp.einsum('atbs,ts->atb', P, f4) / denom2[:, :, None]
    current_cov = jnp.einsum('atb,bij->atij', temp, prev_cov)

    lall, qall = jnp.linalg.eigh(current_cov)
    l_max = lall[:, _NUM_TIME - 1]
    q_max = qall[:, _NUM_TIME - 1]
    left = q_max * jnp.sqrt(l_max)[:, None, :]
    right = qall * (1.0 / jnp.sqrt(lall))[:, :, None, :]
    a = jnp.einsum('yik,ytjk->ytij', left, right)
    return left, right, a


def kernel(prev_x, x, labels, times, P):
    n = x.shape[0]
    segf = times.astype(jnp.int32) * _NUM_LABEL + labels.astype(jnp.int32)
    seg = segf.reshape(n, 1)

    left, right, a = _transform_bank(prev_x, labels, P)

    o2p = _sc_segsum(x, segf)
    o2 = o2p.sum(0)[:_NSEG * _SC_SW].reshape(_NSEG, _SC_SW)
    mu = o2[:, :_D] / jnp.maximum(o2[:, _D], 1.0)[:, None]
    a_seg = jnp.transpose(a, (1, 0, 2, 3)).reshape(_NSEG, _D, _D)
    b_seg = mu - jnp.einsum('sij,sj->si', a_seg, mu)
    w1 = jnp.transpose(a[:, :_SPLIT], (1, 0, 3, 2)).reshape(
        _SPLIT * _NUM_LABEL * _D, _D)
    b_pad = jnp.concatenate(
        [b_seg, jnp.zeros((128 - _NSEG, _D), jnp.float32)], axis=0)
    w_bank = jnp.concatenate([w1, b_pad], axis=0).astype(jnp.bfloat16)

    kw = _SPLIT * _NUM_LABEL * _D + 128
    out = pl.pallas_call(
        _apply_body,
        grid=(n // _RC,),
        in_specs=[
            pl.BlockSpec((_RC, _D), lambda i: (i, 0)),
            pl.BlockSpec((_RC, 1), lambda i: (i, 0)),
            pl.BlockSpec((kw, _D), lambda i: (0, 0)),
        ],
        out_specs=pl.BlockSpec((_RC, _D), lambda i: (i, 0)),
        out_shape=jax.ShapeDtypeStruct((n, _D), jnp.float32),
    )(x, seg, w_bank)
    return out
